# baseline (speedup 1.0000x reference)
import functools

import jax
import jax.numpy as jnp
from jax import lax
from jax.experimental import pallas as pl
from jax.experimental.pallas import tpu as pltpu
from jax.experimental.pallas import tpu_sc as plsc

_LANES = 16


def _make_sc_kernel(batch, maxlen, embed, vocab):
    nc, ns = 2, 16
    nw = nc * ns
    bt = 128
    et = 8
    assert batch % (bt * nw) == 0 and embed % (et * _LANES // 8) == 0
    n_tb = batch // bt
    n_te = embed // et
    k_sl = embed // _LANES
    tb_per_w = n_tb // nw
    assert tb_per_w == 1 and maxlen % 2 == 0

    mesh = plsc.VectorSubcoreMesh(core_axis_name="c", subcore_axis_name="s")

    @functools.partial(
        pl.kernel,
        mesh=mesh,
        compiler_params=pltpu.CompilerParams(use_tc_tiling_on_sc=False,
                                             needs_layout_passes=False),
        out_type=jax.ShapeDtypeStruct((maxlen, n_te, n_tb, et, bt),
                                      jnp.float32),
        scratch_types=[
            pltpu.VMEM((maxlen, bt), jnp.int32),
            pltpu.VMEM((maxlen, embed), jnp.float32),
            [pltpu.VMEM((bt, embed), jnp.float32) for _ in range(4)],
            [pltpu.VMEM((n_te, et, bt + 1), jnp.float32) for _ in range(4)],
            [pltpu.SemaphoreType.DMA for _ in range(4)],
            [pltpu.SemaphoreType.DMA for _ in range(4)],
        ],
    )
    def sc_kernel(xt_hbm, tab_hbm, pos_hbm, out_hbm, idx_v, pos_v, gbuf, sbuf,
                  gsem, ssem):
        cid = lax.axis_index("c")
        sid = lax.axis_index("s")
        w = sid * nc + cid
        pltpu.sync_copy(xt_hbm.at[:, pl.ds(w * bt, bt)], idx_v)
        pltpu.sync_copy(pos_hbm, pos_v)

        lane = lax.broadcasted_iota(jnp.int32, (_LANES,), 0)
        ei_v = lane & 7
        te_vs = [(lane >> 3) + 2 * k for k in range(k_sl)]

        def fire_gather(l, b):
            pltpu.async_copy(tab_hbm.at[idx_v.at[l]], gbuf[b], gsem[b])

        def wait_gather(b):
            pltpu.make_async_copy(tab_hbm.at[pl.ds(0, bt)], gbuf[b],
                                  gsem[b]).wait()

        def fire_store(l, b):
            pltpu.async_copy(sbuf[b].at[:, :, pl.ds(0, bt)],
                             out_hbm.at[l, :, w], ssem[b])

        def wait_store(b):
            pltpu.make_async_copy(sbuf[b].at[:, :, pl.ds(0, bt)],
                                  out_hbm.at[0, :, 0], ssem[b]).wait()

        def transpose_add(l, b):
            pvs = [pos_v[l, pl.ds(k * _LANES, _LANES)] for k in range(k_sl)]

            @plsc.parallel_loop(0, bt, unroll=4)
            def bi_body(bi):
                bsp = jnp.full((_LANES,), 0, jnp.int32) + bi
                for k in range(k_sl):
                    v = gbuf[b][bi, pl.ds(k * _LANES, _LANES)] + pvs[k]
                    plsc.store_scatter(sbuf[b], [te_vs[k], ei_v, bsp], v)

        nbuf = 4
        for b in range(nbuf):
            fire_gather(b, b)
        for b in range(nbuf):
            wait_gather(b)
            transpose_add(b, b)
            fire_gather(b + nbuf, b)
            fire_store(b, b)

        def l_body(t, _):
            l0 = nbuf * t
            for b in range(nbuf):
                l = l0 + b
                wait_gather(b)
                wait_store(b)
                transpose_add(l, b)

                @pl.when(l + nbuf < maxlen)
                def _():
                    fire_gather(l + nbuf, b)

                fire_store(l, b)
            return 0

        lax.fori_loop(1, maxlen // nbuf, l_body, 0, unroll=1)

        for b in range(nbuf):
            wait_store(b)

    return sc_kernel


def kernel(x, token_table, pos_table):
    batch, maxlen = x.shape
    vocab, embed = token_table.shape
    xt = jnp.transpose(x)
    sc = _make_sc_kernel(batch, maxlen, embed, vocab)
    out5 = sc(xt, token_table, pos_table)
    t = out5.transpose(2, 4, 0, 1, 3)
    return t.reshape(batch, maxlen, embed)


# speedup vs baseline: 1.0095x; 1.0095x over previous
import functools

import jax
import jax.numpy as jnp
from jax import lax
from jax.experimental import pallas as pl
from jax.experimental.pallas import tpu as pltpu
from jax.experimental.pallas import tpu_sc as plsc

_LANES = 16


def _make_sc_kernel(batch, maxlen, embed, vocab):
    nc, ns = 2, 16
    nw = nc * ns
    bt = 128
    et = 8
    assert batch % (bt * nw) == 0 and embed % (et * _LANES // 8) == 0
    n_tb = batch // bt
    n_te = embed // et
    k_sl = embed // _LANES
    tb_per_w = n_tb // nw
    assert tb_per_w == 1 and maxlen % 2 == 0

    mesh = plsc.VectorSubcoreMesh(core_axis_name="c", subcore_axis_name="s")

    @functools.partial(
        pl.kernel,
        mesh=mesh,
        compiler_params=pltpu.CompilerParams(use_tc_tiling_on_sc=False,
                                             needs_layout_passes=False),
        out_type=jax.ShapeDtypeStruct((maxlen, n_te, n_tb, et, bt),
                                      jnp.float32),
        scratch_types=[
            pltpu.VMEM((maxlen // et, et, bt), jnp.int32),
            pltpu.VMEM((maxlen, embed), jnp.float32),
            [pltpu.VMEM((bt, embed), jnp.float32) for _ in range(4)],
            [pltpu.VMEM((n_te, et, bt + 1), jnp.float32) for _ in range(4)],
            [pltpu.SemaphoreType.DMA for _ in range(4)],
            [pltpu.SemaphoreType.DMA for _ in range(4)],
        ],
    )
    def sc_kernel(xt_hbm, tab_hbm, pos_hbm, out_hbm, idx_v, pos_v, gbuf, sbuf,
                  gsem, ssem):
        cid = lax.axis_index("c")
        sid = lax.axis_index("s")
        w = sid * nc + cid
        pltpu.sync_copy(xt_hbm.at[:, w], idx_v)
        pltpu.sync_copy(pos_hbm, pos_v)

        lane = lax.broadcasted_iota(jnp.int32, (_LANES,), 0)
        ei_v = lane & 7
        te_vs = [(lane >> 3) + 2 * k for k in range(k_sl)]

        def fire_gather(l, b):
            pltpu.async_copy(tab_hbm.at[idx_v.at[l >> 3, l & 7]], gbuf[b],
                             gsem[b])

        def wait_gather(b):
            pltpu.make_async_copy(tab_hbm.at[pl.ds(0, bt)], gbuf[b],
                                  gsem[b]).wait()

        def fire_store(l, b):
            pltpu.async_copy(sbuf[b].at[:, :, pl.ds(0, bt)],
                             out_hbm.at[l, :, w], ssem[b])

        def wait_store(b):
            pltpu.make_async_copy(sbuf[b].at[:, :, pl.ds(0, bt)],
                                  out_hbm.at[0, :, 0], ssem[b]).wait()

        def transpose_add(l, b):
            pvs = [pos_v[l, pl.ds(k * _LANES, _LANES)] for k in range(k_sl)]

            @plsc.parallel_loop(0, bt, unroll=4)
            def bi_body(bi):
                bsp = jnp.full((_LANES,), 0, jnp.int32) + bi
                for k in range(k_sl):
                    v = gbuf[b][bi, pl.ds(k * _LANES, _LANES)] + pvs[k]
                    plsc.store_scatter(sbuf[b], [te_vs[k], ei_v, bsp], v)

        nbuf = 4
        for b in range(nbuf):
            fire_gather(b, b)
        for b in range(nbuf):
            wait_gather(b)
            transpose_add(b, b)
            fire_gather(b + nbuf, b)
            fire_store(b, b)

        def l_body(t, _):
            l0 = nbuf * t
            for b in range(nbuf):
                l = l0 + b
                wait_gather(b)
                wait_store(b)
                transpose_add(l, b)

                @pl.when(l + nbuf < maxlen)
                def _():
                    fire_gather(l + nbuf, b)

                fire_store(l, b)
            return 0

        lax.fori_loop(1, maxlen // nbuf, l_body, 0, unroll=1)

        for b in range(nbuf):
            wait_store(b)

    return sc_kernel


def kernel(x, token_table, pos_table):
    batch, maxlen = x.shape
    vocab, embed = token_table.shape
    lt, bt = maxlen // 8, batch // 128
    xp = x.reshape(bt, 128, lt, 8).transpose(2, 0, 3, 1)
    sc = _make_sc_kernel(batch, maxlen, embed, vocab)
    out5 = sc(xp, token_table, pos_table)
    t = out5.transpose(2, 4, 0, 1, 3)
    return t.reshape(batch, maxlen, embed)
